# baseline (speedup 1.0000x reference)
import functools

import jax
import jax.numpy as jnp
from jax import lax
from jax.experimental import pallas as pl
from jax.experimental.pallas import tpu as pltpu
from jax.experimental.pallas import tpu_sc as plsc

MOD = 1000000
EMBED = 32
NUM_WORKERS = 32
GATHER = 128
KB = 4
BATCH = GATHER * KB
NBUF = 2

NBLK = 7812
TAIL_COL = NBLK * 128
TAIL_N = 64


def _sc_relayout(table_t, tail_flat):
    mesh = plsc.VectorSubcoreMesh(core_axis_name="c", subcore_axis_name="s")

    @functools.partial(
        pl.kernel,
        mesh=mesh,
        compiler_params=pltpu.CompilerParams(use_tc_tiling_on_sc=True,
                                             needs_layout_passes=False),
        out_type=jax.ShapeDtypeStruct((MOD * EMBED,), jnp.float32),
        scratch_types=[
            pltpu.VMEM((EMBED, 128), jnp.float32),
            pltpu.VMEM((4096,), jnp.float32),
        ],
    )
    def k(tt_hbm, tail_hbm, rm_hbm, blk_v, tp_v):
        wid = lax.axis_index("s") * 2 + lax.axis_index("c")
        lo = wid * jnp.int32(244) + jnp.minimum(wid, jnp.int32(4))
        cnt = jnp.where(wid < jnp.int32(4), jnp.int32(245), jnp.int32(244))
        iota = lax.iota(jnp.int32, 16)

        def transpose_rows(nrows):
            def row_body(r0, carry):
                for dr in range(4):
                    r = r0 * jnp.int32(4) + jnp.int32(dr)
                    rvec = jnp.full((16,), 0, jnp.int32) + r
                    for c0 in (0, 16):
                        v = plsc.load_gather(blk_v, [iota + jnp.int32(c0), rvec])
                        tp_v[pl.ds(r * jnp.int32(EMBED) + jnp.int32(c0), 16)] = v
                return carry

            lax.fori_loop(jnp.int32(0), jnp.int32(nrows // 4), row_body, 0)

        def blk_body(i, carry):
            kblk = lo + i
            pltpu.sync_copy(tt_hbm.at[:, pl.ds(kblk * jnp.int32(128), 128)],
                            blk_v)
            transpose_rows(128)
            pltpu.sync_copy(tp_v,
                            rm_hbm.at[pl.ds(kblk * jnp.int32(4096), 4096)])
            return carry

        lax.fori_loop(jnp.int32(0), cnt, blk_body, 0)

        @pl.when(wid == jnp.int32(NUM_WORKERS - 1))
        def _():
            pltpu.sync_copy(tail_hbm, tp_v.at[pl.ds(0, TAIL_N * EMBED)])
            pltpu.sync_copy(tp_v.at[pl.ds(0, TAIL_N * EMBED)],
                            rm_hbm.at[pl.ds(NBLK * 4096, TAIL_N * EMBED)])

    return k(table_t, tail_flat)


def _sc_embed(xflat, table):
    n = xflat.shape[0]
    per_worker = n // NUM_WORKERS
    num_outer = per_worker // (BATCH * NBUF)
    assert per_worker % (BATCH * NBUF) == 0
    mesh = plsc.VectorSubcoreMesh(core_axis_name="c", subcore_axis_name="s")

    @functools.partial(
        pl.kernel,
        mesh=mesh,
        compiler_params=pltpu.CompilerParams(use_tc_tiling_on_sc=False),
        out_type=jax.ShapeDtypeStruct((n, EMBED), jnp.float32),
        scratch_types=[
            pltpu.VMEM((per_worker,), jnp.int32),
            pltpu.VMEM((NBUF, BATCH, EMBED), jnp.float32),
            pltpu.SemaphoreType.DMA,
            pltpu.SemaphoreType.DMA,
            pltpu.SemaphoreType.DMA,
            pltpu.SemaphoreType.DMA,
        ],
    )
    def k(x_hbm, table_hbm, out_hbm, idx_v, rows_v, sg0, sg1, ss0, ss1):
        sem_g = (sg0, sg1)
        sem_s = (ss0, ss1)
        wid = lax.axis_index("s") * 2 + lax.axis_index("c")
        base = wid * per_worker
        pltpu.sync_copy(x_hbm.at[pl.ds(base, per_worker)], idx_v)

        def mod_body(i, carry):
            o = i * jnp.int32(64)
            for j in range(4):
                v = idx_v[pl.ds(o + jnp.int32(j * 16), 16)]
                idx_v[pl.ds(o + jnp.int32(j * 16), 16)] = lax.rem(
                    v, jnp.int32(MOD))
            return carry

        lax.fori_loop(jnp.int32(0), jnp.int32(per_worker // 64), mod_body, 0)

        def outer_body(t, carry):
            boff = t * jnp.int32(BATCH * NBUF)
            for p in range(NBUF):
                @pl.when(t > jnp.int32(0))
                def _():
                    pltpu.make_async_copy(
                        rows_v.at[jnp.int32(p)], out_hbm.at[pl.ds(base, BATCH)],
                        sem_s[p]).wait()
                off = boff + jnp.int32(p * BATCH)
                for g in range(KB):
                    pltpu.async_copy(
                        table_hbm.at[idx_v.at[pl.ds(off + jnp.int32(g * GATHER),
                                                    GATHER)]],
                        rows_v.at[jnp.int32(p)].at[pl.ds(g * GATHER, GATHER)],
                        sem_g[p],
                    )
            for p in range(NBUF):
                off = boff + jnp.int32(p * BATCH)
                pltpu.make_async_copy(
                    table_hbm.at[idx_v.at[pl.ds(off, GATHER)]],
                    rows_v.at[jnp.int32(p)], sem_g[p]).wait()
                pltpu.async_copy(rows_v.at[jnp.int32(p)],
                                 out_hbm.at[pl.ds(base + off, BATCH)],
                                 sem_s[p])
            return carry

        lax.fori_loop(jnp.int32(0), jnp.int32(num_outer), outer_body, 0)
        for p in range(NBUF):
            pltpu.make_async_copy(
                rows_v.at[jnp.int32(p)], out_hbm.at[pl.ds(base, BATCH)],
                sem_s[p]).wait()

    return k(xflat, table)


def kernel(x, table):
    xflat = x.reshape(-1).astype(jnp.int32)
    tail_flat = table[TAIL_COL:].reshape(-1)
    rm_flat = _sc_relayout(table.T, tail_flat)
    table_rm = rm_flat.reshape(MOD, EMBED)
    out = _sc_embed(xflat, table_rm)
    return out.reshape(x.shape[0], x.shape[1] * EMBED)


# speedup vs baseline: 1.2505x; 1.2505x over previous
import functools

import jax
import jax.numpy as jnp
from jax import lax
from jax.experimental import pallas as pl
from jax.experimental.pallas import tpu as pltpu
from jax.experimental.pallas import tpu_sc as plsc

MOD = 1000000
EMBED = 32
NUM_WORKERS = 32
GATHER = 128
KB = 4
BATCH = GATHER * KB
NBUF = 2

NBLK = 7812
TAIL_COL = NBLK * 128
TAIL_N = 64


SLAB = 512
SLAB_ELEMS = SLAB * EMBED
NSLAB = TAIL_COL // SLAB


def _sc_relayout(table_t, tail_flat):
    mesh = plsc.VectorSubcoreMesh(core_axis_name="c", subcore_axis_name="s")

    @functools.partial(
        pl.kernel,
        mesh=mesh,
        compiler_params=pltpu.CompilerParams(use_tc_tiling_on_sc=True,
                                             needs_layout_passes=False),
        out_type=jax.ShapeDtypeStruct((MOD * EMBED,), jnp.float32),
        scratch_types=[
            pltpu.VMEM((EMBED, SLAB), jnp.float32),
            pltpu.VMEM((EMBED, SLAB), jnp.float32),
            pltpu.VMEM((SLAB_ELEMS,), jnp.float32),
            pltpu.VMEM((SLAB_ELEMS,), jnp.float32),
            pltpu.SemaphoreType.DMA,
            pltpu.SemaphoreType.DMA,
            pltpu.SemaphoreType.DMA,
            pltpu.SemaphoreType.DMA,
        ],
    )
    def k(tt_hbm, tail_hbm, rm_hbm, blk0, blk1, tp0, tp1, sl0, sl1, ss0, ss1):
        blk = (blk0, blk1)
        tp = (tp0, tp1)
        sem_l = (sl0, sl1)
        sem_s = (ss0, ss1)
        wid = lax.axis_index("s") * 2 + lax.axis_index("c")
        lo = wid * jnp.int32(61) + jnp.minimum(wid, jnp.int32(1))
        cnt = jnp.where(wid == jnp.int32(0), jnp.int32(62), jnp.int32(61))
        iota = lax.iota(jnp.int32, 16)

        def transpose(src, dst):
            def row_body(r0, carry):
                for dr in range(8):
                    r = r0 * jnp.int32(8) + jnp.int32(dr)
                    rvec = jnp.full((16,), 0, jnp.int32) + r
                    for c0 in (0, 16):
                        v = plsc.load_gather(src, [iota + jnp.int32(c0), rvec])
                        dst[pl.ds(r * jnp.int32(EMBED) + jnp.int32(c0), 16)] = v
                return carry

            lax.fori_loop(jnp.int32(0), jnp.int32(SLAB // 8), row_body, 0)

        pltpu.async_copy(tt_hbm.at[:, pl.ds(lo * jnp.int32(SLAB), SLAB)],
                         blk0, sl0)

        def outer(t, carry):
            for p in range(2):
                i = t * jnp.int32(2) + jnp.int32(p)

                @pl.when(i < cnt)
                def _():
                    pltpu.make_async_copy(
                        tt_hbm.at[:, pl.ds(0, SLAB)], blk[p], sem_l[p]).wait()

                    @pl.when(i + jnp.int32(1) < cnt)
                    def _():
                        pltpu.async_copy(
                            tt_hbm.at[:, pl.ds((lo + i + 1) * jnp.int32(SLAB),
                                               SLAB)],
                            blk[1 - p], sem_l[1 - p])

                    @pl.when(i >= jnp.int32(2))
                    def _():
                        pltpu.make_async_copy(
                            tp[p], rm_hbm.at[pl.ds(0, SLAB_ELEMS)],
                            sem_s[p]).wait()

                    transpose(blk[p], tp[p])
                    pltpu.async_copy(
                        tp[p],
                        rm_hbm.at[pl.ds((lo + i) * jnp.int32(SLAB_ELEMS),
                                        SLAB_ELEMS)],
                        sem_s[p])
            return carry

        lax.fori_loop(jnp.int32(0), jnp.int32(31), outer, 0)
        for p in range(2):
            pltpu.make_async_copy(
                tp[p], rm_hbm.at[pl.ds(0, SLAB_ELEMS)], sem_s[p]).wait()

        @pl.when(wid == jnp.int32(NUM_WORKERS - 1))
        def _():
            pltpu.sync_copy(tail_hbm, tp0.at[pl.ds(0, TAIL_N * EMBED)])
            pltpu.sync_copy(tp0.at[pl.ds(0, TAIL_N * EMBED)],
                            rm_hbm.at[pl.ds(NBLK * 4096, TAIL_N * EMBED)])

    return k(table_t, tail_flat)


def _sc_embed(xflat, table):
    n = xflat.shape[0]
    per_worker = n // NUM_WORKERS
    num_outer = per_worker // (BATCH * NBUF)
    assert per_worker % (BATCH * NBUF) == 0
    mesh = plsc.VectorSubcoreMesh(core_axis_name="c", subcore_axis_name="s")

    @functools.partial(
        pl.kernel,
        mesh=mesh,
        compiler_params=pltpu.CompilerParams(use_tc_tiling_on_sc=False),
        out_type=jax.ShapeDtypeStruct((n, EMBED), jnp.float32),
        scratch_types=[
            pltpu.VMEM((per_worker,), jnp.int32),
            pltpu.VMEM((NBUF, BATCH, EMBED), jnp.float32),
            pltpu.SemaphoreType.DMA,
            pltpu.SemaphoreType.DMA,
            pltpu.SemaphoreType.DMA,
            pltpu.SemaphoreType.DMA,
        ],
    )
    def k(x_hbm, table_hbm, out_hbm, idx_v, rows_v, sg0, sg1, ss0, ss1):
        sem_g = (sg0, sg1)
        sem_s = (ss0, ss1)
        wid = lax.axis_index("s") * 2 + lax.axis_index("c")
        base = wid * per_worker
        pltpu.sync_copy(x_hbm.at[pl.ds(base, per_worker)], idx_v)

        def mod_body(i, carry):
            o = i * jnp.int32(64)
            for j in range(4):
                v = idx_v[pl.ds(o + jnp.int32(j * 16), 16)]
                idx_v[pl.ds(o + jnp.int32(j * 16), 16)] = lax.rem(
                    v, jnp.int32(MOD))
            return carry

        lax.fori_loop(jnp.int32(0), jnp.int32(per_worker // 64), mod_body, 0)

        def outer_body(t, carry):
            boff = t * jnp.int32(BATCH * NBUF)
            for p in range(NBUF):
                @pl.when(t > jnp.int32(0))
                def _():
                    pltpu.make_async_copy(
                        rows_v.at[jnp.int32(p)], out_hbm.at[pl.ds(base, BATCH)],
                        sem_s[p]).wait()
                off = boff + jnp.int32(p * BATCH)
                for g in range(KB):
                    pltpu.async_copy(
                        table_hbm.at[idx_v.at[pl.ds(off + jnp.int32(g * GATHER),
                                                    GATHER)]],
                        rows_v.at[jnp.int32(p)].at[pl.ds(g * GATHER, GATHER)],
                        sem_g[p],
                    )
            for p in range(NBUF):
                off = boff + jnp.int32(p * BATCH)
                pltpu.make_async_copy(
                    table_hbm.at[idx_v.at[pl.ds(off, GATHER)]],
                    rows_v.at[jnp.int32(p)], sem_g[p]).wait()
                pltpu.async_copy(rows_v.at[jnp.int32(p)],
                                 out_hbm.at[pl.ds(base + off, BATCH)],
                                 sem_s[p])
            return carry

        lax.fori_loop(jnp.int32(0), jnp.int32(num_outer), outer_body, 0)
        for p in range(NBUF):
            pltpu.make_async_copy(
                rows_v.at[jnp.int32(p)], out_hbm.at[pl.ds(base, BATCH)],
                sem_s[p]).wait()

    return k(xflat, table)


def kernel(x, table):
    xflat = x.reshape(-1).astype(jnp.int32)
    tail_flat = table[TAIL_COL:].reshape(-1)
    rm_flat = _sc_relayout(table.T, tail_flat)
    table_rm = rm_flat.reshape(MOD, EMBED)
    out = _sc_embed(xflat, table_rm)
    return out.reshape(x.shape[0], x.shape[1] * EMBED)


# speedup vs baseline: 1.4704x; 1.1758x over previous
import functools

import jax
import jax.numpy as jnp
from jax import lax
from jax.experimental import pallas as pl
from jax.experimental.pallas import tpu as pltpu
from jax.experimental.pallas import tpu_sc as plsc

MOD = 1000000
EMBED = 32
NUM_WORKERS = 32
GATHER = 128
KB = 4
BATCH = GATHER * KB
NBUF = 2

NBLK = 7812
TAIL_COL = NBLK * 128
TAIL_N = 64


SLAB = 512
SLAB_ELEMS = SLAB * EMBED
NSLAB = TAIL_COL // SLAB


def _sc_relayout(table_t, tail_flat):
    mesh = plsc.VectorSubcoreMesh(core_axis_name="c", subcore_axis_name="s")

    @functools.partial(
        pl.kernel,
        mesh=mesh,
        compiler_params=pltpu.CompilerParams(use_tc_tiling_on_sc=True,
                                             needs_layout_passes=False),
        out_type=jax.ShapeDtypeStruct((MOD * EMBED,), jnp.float32),
        scratch_types=[
            pltpu.VMEM((EMBED, SLAB), jnp.float32),
            pltpu.VMEM((EMBED, SLAB), jnp.float32),
            pltpu.VMEM((SLAB_ELEMS,), jnp.float32),
            pltpu.VMEM((SLAB_ELEMS,), jnp.float32),
            pltpu.SemaphoreType.DMA,
            pltpu.SemaphoreType.DMA,
            pltpu.SemaphoreType.DMA,
            pltpu.SemaphoreType.DMA,
        ],
    )
    def k(tt_hbm, tail_hbm, rm_hbm, blk0, blk1, tp0, tp1, sl0, sl1, ss0, ss1):
        blk = (blk0, blk1)
        tp = (tp0, tp1)
        sem_l = (sl0, sl1)
        sem_s = (ss0, ss1)
        wid = lax.axis_index("s") * 2 + lax.axis_index("c")
        lo = wid * jnp.int32(61) + jnp.minimum(wid, jnp.int32(1))
        cnt = jnp.where(wid == jnp.int32(0), jnp.int32(62), jnp.int32(61))
        iota = lax.iota(jnp.int32, 16)

        iota32 = iota * jnp.int32(EMBED)

        def transpose(src, dst):
            def row_body(r0i, carry):
                r0 = r0i * jnp.int32(16)
                dbase = r0 * jnp.int32(EMBED)
                for c in range(EMBED):
                    v = src[jnp.int32(c), pl.ds(r0, 16)]
                    plsc.store_scatter(dst, [iota32 + (dbase + jnp.int32(c))],
                                       v)
                return carry

            lax.fori_loop(jnp.int32(0), jnp.int32(SLAB // 16), row_body, 0)

        pltpu.async_copy(tt_hbm.at[:, pl.ds(lo * jnp.int32(SLAB), SLAB)],
                         blk0, sl0)

        def outer(t, carry):
            for p in range(2):
                i = t * jnp.int32(2) + jnp.int32(p)

                @pl.when(i < cnt)
                def _():
                    pltpu.make_async_copy(
                        tt_hbm.at[:, pl.ds(0, SLAB)], blk[p], sem_l[p]).wait()

                    @pl.when(i + jnp.int32(1) < cnt)
                    def _():
                        pltpu.async_copy(
                            tt_hbm.at[:, pl.ds((lo + i + 1) * jnp.int32(SLAB),
                                               SLAB)],
                            blk[1 - p], sem_l[1 - p])

                    @pl.when(i >= jnp.int32(2))
                    def _():
                        pltpu.make_async_copy(
                            tp[p], rm_hbm.at[pl.ds(0, SLAB_ELEMS)],
                            sem_s[p]).wait()

                    transpose(blk[p], tp[p])
                    pltpu.async_copy(
                        tp[p],
                        rm_hbm.at[pl.ds((lo + i) * jnp.int32(SLAB_ELEMS),
                                        SLAB_ELEMS)],
                        sem_s[p])
            return carry

        lax.fori_loop(jnp.int32(0), jnp.int32(31), outer, 0)
        for p in range(2):
            pltpu.make_async_copy(
                tp[p], rm_hbm.at[pl.ds(0, SLAB_ELEMS)], sem_s[p]).wait()

        @pl.when(wid == jnp.int32(NUM_WORKERS - 1))
        def _():
            pltpu.sync_copy(tail_hbm, tp0.at[pl.ds(0, TAIL_N * EMBED)])
            pltpu.sync_copy(tp0.at[pl.ds(0, TAIL_N * EMBED)],
                            rm_hbm.at[pl.ds(NBLK * 4096, TAIL_N * EMBED)])

    return k(table_t, tail_flat)


def _sc_embed(xflat, table):
    n = xflat.shape[0]
    per_worker = n // NUM_WORKERS
    num_outer = per_worker // (BATCH * NBUF)
    assert per_worker % (BATCH * NBUF) == 0
    mesh = plsc.VectorSubcoreMesh(core_axis_name="c", subcore_axis_name="s")

    @functools.partial(
        pl.kernel,
        mesh=mesh,
        compiler_params=pltpu.CompilerParams(use_tc_tiling_on_sc=False),
        out_type=jax.ShapeDtypeStruct((n, EMBED), jnp.float32),
        scratch_types=[
            pltpu.VMEM((per_worker,), jnp.int32),
            pltpu.VMEM((NBUF, BATCH, EMBED), jnp.float32),
            pltpu.SemaphoreType.DMA,
            pltpu.SemaphoreType.DMA,
            pltpu.SemaphoreType.DMA,
            pltpu.SemaphoreType.DMA,
        ],
    )
    def k(x_hbm, table_hbm, out_hbm, idx_v, rows_v, sg0, sg1, ss0, ss1):
        sem_g = (sg0, sg1)
        sem_s = (ss0, ss1)
        wid = lax.axis_index("s") * 2 + lax.axis_index("c")
        base = wid * per_worker
        pltpu.sync_copy(x_hbm.at[pl.ds(base, per_worker)], idx_v)

        def mod_body(i, carry):
            o = i * jnp.int32(64)
            for j in range(4):
                v = idx_v[pl.ds(o + jnp.int32(j * 16), 16)]
                idx_v[pl.ds(o + jnp.int32(j * 16), 16)] = lax.rem(
                    v, jnp.int32(MOD))
            return carry

        lax.fori_loop(jnp.int32(0), jnp.int32(per_worker // 64), mod_body, 0)

        def outer_body(t, carry):
            boff = t * jnp.int32(BATCH * NBUF)
            for p in range(NBUF):
                @pl.when(t > jnp.int32(0))
                def _():
                    pltpu.make_async_copy(
                        rows_v.at[jnp.int32(p)], out_hbm.at[pl.ds(base, BATCH)],
                        sem_s[p]).wait()
                off = boff + jnp.int32(p * BATCH)
                for g in range(KB):
                    pltpu.async_copy(
                        table_hbm.at[idx_v.at[pl.ds(off + jnp.int32(g * GATHER),
                                                    GATHER)]],
                        rows_v.at[jnp.int32(p)].at[pl.ds(g * GATHER, GATHER)],
                        sem_g[p],
                    )
            for p in range(NBUF):
                off = boff + jnp.int32(p * BATCH)
                pltpu.make_async_copy(
                    table_hbm.at[idx_v.at[pl.ds(off, GATHER)]],
                    rows_v.at[jnp.int32(p)], sem_g[p]).wait()
                pltpu.async_copy(rows_v.at[jnp.int32(p)],
                                 out_hbm.at[pl.ds(base + off, BATCH)],
                                 sem_s[p])
            return carry

        lax.fori_loop(jnp.int32(0), jnp.int32(num_outer), outer_body, 0)
        for p in range(NBUF):
            pltpu.make_async_copy(
                rows_v.at[jnp.int32(p)], out_hbm.at[pl.ds(base, BATCH)],
                sem_s[p]).wait()

    return k(xflat, table)


def kernel(x, table):
    xflat = x.reshape(-1).astype(jnp.int32)
    tail_flat = table[TAIL_COL:].reshape(-1)
    rm_flat = _sc_relayout(table.T, tail_flat)
    table_rm = rm_flat.reshape(MOD, EMBED)
    out = _sc_embed(xflat, table_rm)
    return out.reshape(x.shape[0], x.shape[1] * EMBED)
